# baseline (speedup 1.0000x reference)
import functools

import jax
import jax.numpy as jnp
from jax import lax
from jax.experimental import pallas as pl
from jax.experimental.pallas import tpu as pltpu
from jax.experimental.pallas import tpu_sc as plsc

NC = 2
NS = 16
NW = NC * NS
L = 16

B, T, H, W = 32, 21, 256, 256
HR = H // 2
HALF = HR * W
VPR = W // L
NACC = 2
UNROLL = 1


def _merge(va, ja, vb, jb):
    m = (vb > va) | ((vb == va) & (jb < ja))
    return jnp.where(m, vb, va), jnp.where(m, jb, ja)


def _scan_half(buf, joff):
    neg_inf = jnp.full((L,), -jnp.inf, dtype=jnp.float32)
    zeros = jnp.zeros((L,), dtype=jnp.int32)

    init = tuple((neg_inf, zeros) for _ in range(NACC))

    @pl.loop(0, HR, init_carry=init, unroll=UNROLL)
    def scan(r, carry):
        pairs = list(carry)
        jbase = joff + r * VPR
        for k in range(VPR):
            vacc, jacc = pairs[k % NACC]
            v = buf[r, pl.ds(k * L, L)]
            m = v > vacc
            vacc = jnp.where(m, v, vacc)
            jacc = jnp.where(m, jbase + k, jacc)
            pairs[k % NACC] = (vacc, jacc)
        return tuple(pairs)

    pairs = list(scan)
    while len(pairs) > 1:
        merged = []
        for i in range(0, len(pairs), 2):
            (va, ja), (vb, jb) = pairs[i], pairs[i + 1]
            merged.append(_merge(va, ja, vb, jb))
        pairs = merged
    return pairs[0]


def _lane_argmax(vacc, jacc):
    mval = vacc[0]
    for k in range(1, L):
        mval = jnp.maximum(mval, vacc[k])
    idx = jnp.int32(1 << 30)
    for k in range(L):
        lin = jacc[k] * L + k
        idx = jnp.where(vacc[k] == mval, jnp.minimum(idx, lin), idx)
    return idx


def _sc_argmax(data, t_lo, t_cnt):
    mesh = plsc.VectorSubcoreMesh(
        core_axis_name="c", subcore_axis_name="s", num_cores=NC, num_subcores=NS
    )

    @functools.partial(
        pl.kernel,
        out_type=jax.ShapeDtypeStruct((B, t_cnt, L), jnp.int32),
        mesh=mesh,
        scratch_types=[
            pltpu.VMEM((HR, W), jnp.float32),
            pltpu.VMEM((HR, W), jnp.float32),
            pltpu.VMEM((t_cnt, L), jnp.int32),
            pltpu.SemaphoreType.DMA,
            pltpu.SemaphoreType.DMA,
        ],
    )
    def kern(x_hbm, out_hbm, buf0, buf1, stage, sem0, sem1):
        wid = lax.axis_index("s") * NC + lax.axis_index("c")

        pltpu.async_copy(x_hbm.at[wid, t_lo, pl.ds(0, HR)], buf0, sem0)

        @pl.loop(0, t_cnt)
        def per_map(tt):
            t = t_lo + tt
            pltpu.async_copy(x_hbm.at[wid, t, pl.ds(HR, HR)], buf1, sem1)
            pltpu.make_async_copy(x_hbm.at[wid, t, pl.ds(0, HR)], buf0, sem0).wait()
            vacc0, jacc0 = _scan_half(buf0, 0)

            @pl.when(tt < t_cnt - 1)
            def _():
                pltpu.async_copy(x_hbm.at[wid, t + 1, pl.ds(0, HR)], buf0, sem0)

            pltpu.make_async_copy(x_hbm.at[wid, t, pl.ds(HR, HR)], buf1, sem1).wait()
            vacc1, jacc1 = _scan_half(buf1, HR * VPR)

            m = vacc1 > vacc0
            vacc = jnp.where(m, vacc1, vacc0)
            jacc = jnp.where(m, jacc1, jacc0)
            idx = _lane_argmax(vacc, jacc)
            xcoord = jnp.remainder(idx, W)
            ycoord = idx // W
            lanes = lax.iota(jnp.int32, L)
            row = jnp.where(
                lanes == 0, xcoord, jnp.where(lanes == 1, ycoord, 0)
            ).astype(jnp.int32)
            stage[tt, :] = row

        pltpu.sync_copy(stage, out_hbm.at[wid])

    return kern(data)


T_TC = 6


TC_RB = 8
TC_NCH = H // TC_RB


def _tc_block_body(x_ref, out_ref):
    pos = (
        lax.broadcasted_iota(jnp.int32, (TC_RB, W), 0) * W
        + lax.broadcasted_iota(jnp.int32, (TC_RB, W), 1)
    )
    big = jnp.full((TC_RB, W), jnp.int32(1 << 30))
    for t in range(T_TC):
        accs = []
        for q in range(2):
            v = x_ref[0, t, pl.ds(TC_RB * q, TC_RB), :]
            accs.append([v, jnp.full((TC_RB, W), jnp.int32(q))])
        for i in range(2, TC_NCH):
            v = x_ref[0, t, pl.ds(TC_RB * i, TC_RB), :]
            vacc, iacc = accs[i % 2]
            m = v > vacc
            accs[i % 2] = [
                jnp.where(m, v, vacc),
                jnp.where(m, jnp.int32(i), iacc),
            ]
        (v0, i0), (v1, i1) = accs
        mg = (v1 > v0) | ((v1 == v0) & (i1 < i0))
        vacc = jnp.where(mg, v1, v0)
        iacc = jnp.where(mg, i1, i0)
        mval = jnp.max(vacc)
        lin = iacc * (TC_RB * W) + pos
        cand = jnp.where(vacc == mval, lin, big)
        idx = jnp.min(cand)
        out_ref[0, t, 0, :] = jnp.stack([idx % W, idx // W]).reshape(2)


def _tc_argmax(data):
    return pl.pallas_call(
        _tc_block_body,
        grid=(B,),
        in_specs=[pl.BlockSpec((1, T_TC, H, W), lambda b: (b, 0, 0, 0))],
        out_specs=pl.BlockSpec((1, T_TC, 1, 2), lambda b: (b, 0, 0, 0)),
        out_shape=jax.ShapeDtypeStruct((B, T_TC, 1, 2), jnp.int32),
    )(data)


@jax.jit
def kernel(data):
    sc_out = _sc_argmax(data, T_TC, T - T_TC)
    tc_out = _tc_argmax(data)
    coords = jnp.concatenate(
        [tc_out.reshape(B, T_TC, 2), sc_out[:, :, :2]], axis=1
    )
    return coords.astype(jnp.int64)


# speedup vs baseline: 1.0006x; 1.0006x over previous
import functools

import jax
import jax.numpy as jnp
from jax import lax
from jax.experimental import pallas as pl
from jax.experimental.pallas import tpu as pltpu
from jax.experimental.pallas import tpu_sc as plsc

NC = 2
NS = 16
NW = NC * NS
L = 16

B, T, H, W = 32, 21, 256, 256
HR = H // 2
HALF = HR * W
VPR = W // L
NACC = 2
UNROLL = 1


def _merge(va, ja, vb, jb):
    m = (vb > va) | ((vb == va) & (jb < ja))
    return jnp.where(m, vb, va), jnp.where(m, jb, ja)


def _scan_half(buf, joff):
    neg_inf = jnp.full((L,), -jnp.inf, dtype=jnp.float32)
    zeros = jnp.zeros((L,), dtype=jnp.int32)

    init = tuple((neg_inf, zeros) for _ in range(NACC))

    @pl.loop(0, HR, init_carry=init, unroll=UNROLL)
    def scan(r, carry):
        pairs = list(carry)
        jbase = joff + r * VPR
        for k in range(VPR):
            vacc, jacc = pairs[k % NACC]
            v = buf[r, pl.ds(k * L, L)]
            m = v > vacc
            vacc = jnp.where(m, v, vacc)
            jacc = jnp.where(m, jbase + k, jacc)
            pairs[k % NACC] = (vacc, jacc)
        return tuple(pairs)

    pairs = list(scan)
    while len(pairs) > 1:
        merged = []
        for i in range(0, len(pairs), 2):
            (va, ja), (vb, jb) = pairs[i], pairs[i + 1]
            merged.append(_merge(va, ja, vb, jb))
        pairs = merged
    return pairs[0]


def _lane_argmax(vacc, jacc):
    mval = vacc[0]
    for k in range(1, L):
        mval = jnp.maximum(mval, vacc[k])
    idx = jnp.int32(1 << 30)
    for k in range(L):
        lin = jacc[k] * L + k
        idx = jnp.where(vacc[k] == mval, jnp.minimum(idx, lin), idx)
    return idx


def _sc_argmax(data, t_lo, t_cnt):
    mesh = plsc.VectorSubcoreMesh(
        core_axis_name="c", subcore_axis_name="s", num_cores=NC, num_subcores=NS
    )

    @functools.partial(
        pl.kernel,
        out_type=jax.ShapeDtypeStruct((B, t_cnt, L), jnp.int32),
        mesh=mesh,
        scratch_types=[
            pltpu.VMEM((HR, W), jnp.float32),
            pltpu.VMEM((HR, W), jnp.float32),
            pltpu.VMEM((t_cnt, L), jnp.int32),
            pltpu.SemaphoreType.DMA,
            pltpu.SemaphoreType.DMA,
        ],
    )
    def kern(x_hbm, out_hbm, buf0, buf1, stage, sem0, sem1):
        wid = lax.axis_index("s") * NC + lax.axis_index("c")

        pltpu.async_copy(x_hbm.at[wid, t_lo, pl.ds(0, HR)], buf0, sem0)

        @pl.loop(0, t_cnt)
        def per_map(tt):
            t = t_lo + tt
            pltpu.async_copy(x_hbm.at[wid, t, pl.ds(HR, HR)], buf1, sem1)
            pltpu.make_async_copy(x_hbm.at[wid, t, pl.ds(0, HR)], buf0, sem0).wait()
            vacc0, jacc0 = _scan_half(buf0, 0)

            @pl.when(tt < t_cnt - 1)
            def _():
                pltpu.async_copy(x_hbm.at[wid, t + 1, pl.ds(0, HR)], buf0, sem0)

            pltpu.make_async_copy(x_hbm.at[wid, t, pl.ds(HR, HR)], buf1, sem1).wait()
            vacc1, jacc1 = _scan_half(buf1, HR * VPR)

            m = vacc1 > vacc0
            vacc = jnp.where(m, vacc1, vacc0)
            jacc = jnp.where(m, jacc1, jacc0)
            idx = _lane_argmax(vacc, jacc)
            xcoord = jnp.remainder(idx, W)
            ycoord = idx // W
            lanes = lax.iota(jnp.int32, L)
            row = jnp.where(
                lanes == 0, xcoord, jnp.where(lanes == 1, ycoord, 0)
            ).astype(jnp.int32)
            stage[tt, :] = row

        pltpu.sync_copy(stage, out_hbm.at[wid])

    return kern(data)


T_TC = 6


TC_RB = 8
TC_NCH = H // TC_RB


def _tc_block_body(x_ref, out_ref):
    pos = (
        lax.broadcasted_iota(jnp.int32, (TC_RB, W), 0) * W
        + lax.broadcasted_iota(jnp.int32, (TC_RB, W), 1)
    )
    big = jnp.full((TC_RB, W), jnp.int32(1 << 30))

    def run_maps(ts):
        accs = {}
        for t in ts:
            accs[t] = [
                [
                    x_ref[0, t, pl.ds(TC_RB * q, TC_RB), :],
                    jnp.full((TC_RB, W), jnp.int32(q)),
                ]
                for q in range(2)
            ]
        for i in range(2, TC_NCH):
            for t in ts:
                v = x_ref[0, t, pl.ds(TC_RB * i, TC_RB), :]
                vacc, iacc = accs[t][i % 2]
                m = v > vacc
                accs[t][i % 2] = [
                    jnp.where(m, v, vacc),
                    jnp.where(m, jnp.int32(i), iacc),
                ]
        for t in ts:
            (v0, i0), (v1, i1) = accs[t]
            mg = (v1 > v0) | ((v1 == v0) & (i1 < i0))
            vacc = jnp.where(mg, v1, v0)
            iacc = jnp.where(mg, i1, i0)
            mval = jnp.max(vacc)
            lin = iacc * (TC_RB * W) + pos
            cand = jnp.where(vacc == mval, lin, big)
            idx = jnp.min(cand)
            out_ref[0, t, 0, :] = jnp.stack([idx % W, idx // W]).reshape(2)

    for p in range(0, T_TC - 1, 2):
        run_maps([p, p + 1])
    if T_TC % 2:
        run_maps([T_TC - 1])


def _tc_argmax(data):
    return pl.pallas_call(
        _tc_block_body,
        grid=(B,),
        in_specs=[pl.BlockSpec((1, T_TC, H, W), lambda b: (b, 0, 0, 0))],
        out_specs=pl.BlockSpec((1, T_TC, 1, 2), lambda b: (b, 0, 0, 0)),
        out_shape=jax.ShapeDtypeStruct((B, T_TC, 1, 2), jnp.int32),
    )(data)


@jax.jit
def kernel(data):
    sc_out = _sc_argmax(data, T_TC, T - T_TC)
    tc_out = _tc_argmax(data)
    coords = jnp.concatenate(
        [tc_out.reshape(B, T_TC, 2), sc_out[:, :, :2]], axis=1
    )
    return coords.astype(jnp.int64)


# speedup vs baseline: 1.0070x; 1.0064x over previous
import functools

import jax
import jax.numpy as jnp
from jax import lax
from jax.experimental import pallas as pl
from jax.experimental.pallas import tpu as pltpu
from jax.experimental.pallas import tpu_sc as plsc

NC = 2
NS = 16
NW = NC * NS
L = 16

B, T, H, W = 32, 21, 256, 256
HR = H // 2
HALF = HR * W
VPR = W // L
NACC = 2
UNROLL = 1


def _merge(va, ja, vb, jb):
    m = (vb > va) | ((vb == va) & (jb < ja))
    return jnp.where(m, vb, va), jnp.where(m, jb, ja)


def _scan_half(buf, joff):
    neg_inf = jnp.full((L,), -jnp.inf, dtype=jnp.float32)
    zeros = jnp.zeros((L,), dtype=jnp.int32)

    init = tuple((neg_inf, zeros) for _ in range(NACC))

    @pl.loop(0, HR, init_carry=init, unroll=UNROLL)
    def scan(r, carry):
        pairs = list(carry)
        jbase = joff + r * VPR
        for k in range(VPR):
            vacc, jacc = pairs[k % NACC]
            v = buf[r, pl.ds(k * L, L)]
            m = v > vacc
            vacc = jnp.where(m, v, vacc)
            jacc = jnp.where(m, jbase + k, jacc)
            pairs[k % NACC] = (vacc, jacc)
        return tuple(pairs)

    pairs = list(scan)
    while len(pairs) > 1:
        merged = []
        for i in range(0, len(pairs), 2):
            (va, ja), (vb, jb) = pairs[i], pairs[i + 1]
            merged.append(_merge(va, ja, vb, jb))
        pairs = merged
    return pairs[0]


def _lane_argmax(vacc, jacc):
    mval = vacc[0]
    for k in range(1, L):
        mval = jnp.maximum(mval, vacc[k])
    idx = jnp.int32(1 << 30)
    for k in range(L):
        lin = jacc[k] * L + k
        idx = jnp.where(vacc[k] == mval, jnp.minimum(idx, lin), idx)
    return idx


def _sc_argmax(data, t_lo, t_cnt):
    mesh = plsc.VectorSubcoreMesh(
        core_axis_name="c", subcore_axis_name="s", num_cores=NC, num_subcores=NS
    )

    @functools.partial(
        pl.kernel,
        out_type=jax.ShapeDtypeStruct((B, t_cnt, L), jnp.int32),
        mesh=mesh,
        scratch_types=[
            pltpu.VMEM((HR, W), jnp.float32),
            pltpu.VMEM((HR, W), jnp.float32),
            pltpu.VMEM((t_cnt, L), jnp.int32),
            pltpu.SemaphoreType.DMA,
            pltpu.SemaphoreType.DMA,
        ],
    )
    def kern(x_hbm, out_hbm, buf0, buf1, stage, sem0, sem1):
        wid = lax.axis_index("s") * NC + lax.axis_index("c")

        pltpu.async_copy(x_hbm.at[wid, t_lo, pl.ds(0, HR)], buf0, sem0)

        @pl.loop(0, t_cnt)
        def per_map(tt):
            t = t_lo + tt
            pltpu.async_copy(x_hbm.at[wid, t, pl.ds(HR, HR)], buf1, sem1)
            pltpu.make_async_copy(x_hbm.at[wid, t, pl.ds(0, HR)], buf0, sem0).wait()
            vacc0, jacc0 = _scan_half(buf0, 0)

            @pl.when(tt < t_cnt - 1)
            def _():
                pltpu.async_copy(x_hbm.at[wid, t + 1, pl.ds(0, HR)], buf0, sem0)

            pltpu.make_async_copy(x_hbm.at[wid, t, pl.ds(HR, HR)], buf1, sem1).wait()
            vacc1, jacc1 = _scan_half(buf1, HR * VPR)

            m = vacc1 > vacc0
            vacc = jnp.where(m, vacc1, vacc0)
            jacc = jnp.where(m, jacc1, jacc0)
            idx = _lane_argmax(vacc, jacc)
            xcoord = jnp.remainder(idx, W)
            ycoord = idx // W
            lanes = lax.iota(jnp.int32, L)
            row = jnp.where(
                lanes == 0, xcoord, jnp.where(lanes == 1, ycoord, 0)
            ).astype(jnp.int32)
            stage[tt, :] = row

        pltpu.sync_copy(stage, out_hbm.at[wid])

    return kern(data)


T_TC = 7


TC_RB = 8
TC_NCH = H // TC_RB


def _tc_block_body(x_ref, out_ref):
    pos = (
        lax.broadcasted_iota(jnp.int32, (TC_RB, W), 0) * W
        + lax.broadcasted_iota(jnp.int32, (TC_RB, W), 1)
    )
    big = jnp.full((TC_RB, W), jnp.int32(1 << 30))

    def run_maps(ts):
        accs = {}
        for t in ts:
            accs[t] = [
                [
                    x_ref[0, t, pl.ds(TC_RB * q, TC_RB), :],
                    jnp.full((TC_RB, W), jnp.int32(q)),
                ]
                for q in range(2)
            ]
        for i in range(2, TC_NCH):
            for t in ts:
                v = x_ref[0, t, pl.ds(TC_RB * i, TC_RB), :]
                vacc, iacc = accs[t][i % 2]
                m = v > vacc
                accs[t][i % 2] = [
                    jnp.where(m, v, vacc),
                    jnp.where(m, jnp.int32(i), iacc),
                ]
        for t in ts:
            (v0, i0), (v1, i1) = accs[t]
            mg = (v1 > v0) | ((v1 == v0) & (i1 < i0))
            vacc = jnp.where(mg, v1, v0)
            iacc = jnp.where(mg, i1, i0)
            mval = jnp.max(vacc)
            lin = iacc * (TC_RB * W) + pos
            cand = jnp.where(vacc == mval, lin, big)
            idx = jnp.min(cand)
            out_ref[0, t, 0, :] = jnp.stack([idx % W, idx // W]).reshape(2)

    for p in range(0, T_TC - 1, 2):
        run_maps([p, p + 1])
    if T_TC % 2:
        run_maps([T_TC - 1])


def _tc_argmax(data):
    return pl.pallas_call(
        _tc_block_body,
        grid=(B,),
        in_specs=[pl.BlockSpec((1, T_TC, H, W), lambda b: (b, 0, 0, 0))],
        out_specs=pl.BlockSpec((1, T_TC, 1, 2), lambda b: (b, 0, 0, 0)),
        out_shape=jax.ShapeDtypeStruct((B, T_TC, 1, 2), jnp.int32),
    )(data)


@jax.jit
def kernel(data):
    sc_out = _sc_argmax(data, T_TC, T - T_TC)
    tc_out = _tc_argmax(data)
    coords = jnp.concatenate(
        [tc_out.reshape(B, T_TC, 2), sc_out[:, :, :2]], axis=1
    )
    return coords.astype(jnp.int64)
